# Initial kernel scaffold; baseline (speedup 1.0000x reference)
import jax
import jax.numpy as jnp
from jax.experimental import pallas as pl


def kernel(drug_feature, drug_adj, ibatch, gexpr_data, W1, b1, g1, beta1, W2, b2, g2, beta2, Wc1, bc1, gc1, betac1, Wc2, bc2):
    raise NotImplementedError("write your pallas kernel here")


# speedup vs baseline: 9.4405x; 9.4405x over previous
import functools

import jax
import jax.numpy as jnp
from jax import lax
from jax.experimental import pallas as pl
from jax.experimental.pallas import tpu as pltpu
from jax.experimental.pallas import tpu_sc as plsc

N = 10000
E = 320000
D_DRUG = 128
D_CELL = 954
HID = 128
OUT = 256
B = 128

NC = 2
NS = 16
CH = 80
NP = 10240
RPS = NP // NS

@functools.cache
def _mesh():
    return plsc.VectorSubcoreMesh(core_axis_name="c", subcore_axis_name="s")


def _sc_conv(y, src_idx, dst_idx, *, edge_core_stride, per_core_edges,
             idx_core_stride, init_core_stride):
    epb = per_core_edges // NS
    nchunks = epb // CH

    @functools.partial(
        pl.kernel,
        out_type=jax.ShapeDtypeStruct((NC, NP, HID), jnp.float32),
        mesh=_mesh(),
        scratch_types=[
            pltpu.VMEM_SHARED((NP, HID), jnp.float32),
            pltpu.VMEM((CH,), jnp.int32),
            pltpu.VMEM((CH,), jnp.int32),
            pltpu.VMEM((CH, HID), jnp.float32),
            pltpu.SemaphoreType.DMA,
        ],
    )
    def k(y_hbm, src_hbm, dst_hbm, out_hbm, acc_sh, sidx, didx, rows, sem):
        c = lax.axis_index("c")
        s = lax.axis_index("s")
        r0 = s * RPS
        pltpu.sync_copy(y_hbm.at[pl.ds(init_core_stride * c + r0, RPS)],
                        acc_sh.at[pl.ds(r0, RPS)])
        plsc.subcore_barrier()

        @pl.loop(0, nchunks)
        def _(kk):
            e = edge_core_stride * c + s * epb + kk * CH
            pltpu.sync_copy(src_hbm.at[pl.ds(idx_core_stride * c + e, CH)], sidx)
            pltpu.sync_copy(dst_hbm.at[pl.ds(e, CH)], didx)
            pltpu.async_copy(y_hbm.at[sidx], rows, sem).wait()
            pltpu.sync_copy(rows, acc_sh.at[didx], add=True)

        plsc.subcore_barrier()
        pltpu.sync_copy(acc_sh.at[pl.ds(r0, RPS)],
                        out_hbm.at[c].at[pl.ds(r0, RPS)])

    return k(y, src_idx, dst_idx)


def _sc_degree(zeros_init, ones_rows, dst_idx):
    epb = E // NC // NS
    nchunks = epb // CH

    @functools.partial(
        pl.kernel,
        out_type=jax.ShapeDtypeStruct((NC, NP, HID), jnp.float32),
        mesh=_mesh(),
        scratch_types=[
            pltpu.VMEM_SHARED((NP, HID), jnp.float32),
            pltpu.VMEM((CH,), jnp.int32),
            pltpu.VMEM((CH, HID), jnp.float32),
            pltpu.SemaphoreType.DMA,
        ],
    )
    def k(z_hbm, ones_hbm, dst_hbm, out_hbm, acc_sh, didx, ones_v, sem):
        c = lax.axis_index("c")
        s = lax.axis_index("s")
        r0 = s * RPS
        pltpu.sync_copy(z_hbm, acc_sh.at[pl.ds(r0, RPS)])
        pltpu.sync_copy(ones_hbm, ones_v)
        plsc.subcore_barrier()

        @pl.loop(0, nchunks)
        def _(kk):
            e = c * (E // NC) + s * epb + kk * CH
            pltpu.sync_copy(dst_hbm.at[pl.ds(e, CH)], didx)
            pltpu.sync_copy(ones_v, acc_sh.at[didx], add=True)

        plsc.subcore_barrier()
        pltpu.sync_copy(acc_sh.at[pl.ds(r0, RPS)],
                        out_hbm.at[c].at[pl.ds(r0, RPS)])

    return k(zeros_init, ones_rows, dst_idx)


_DOT = dict(preferred_element_type=jnp.float32, precision=lax.Precision.HIGHEST)


def _mm_body(x_ref, w_ref, o_ref):
    o_ref[...] = jnp.dot(x_ref[...], w_ref[...], **_DOT)


def _tc_xw1(x, w):
    nb = 16
    return pl.pallas_call(
        _mm_body,
        grid=(nb,),
        in_specs=[pl.BlockSpec((NP // nb, D_DRUG), lambda i: (i, 0)),
                  pl.BlockSpec((D_DRUG, HID), lambda i: (0, 0))],
        out_specs=pl.BlockSpec((NP // nb, HID), lambda i: (i, 0)),
        out_shape=jax.ShapeDtypeStruct((NP, HID), jnp.float32),
    )(x, w)


def _scale_body(dega_ref, xw_ref, y_ref, dinv_ref):
    deg = dega_ref[0] + dega_ref[1] + 1.0
    dinv = lax.rsqrt(jnp.maximum(deg, 1e-12))
    dinv_ref[...] = dinv[:, 0:16]
    y_ref[...] = xw_ref[...] * dinv[:, 0:1]


def _tc_scale(dega, xw1):
    nb = 16
    return pl.pallas_call(
        _scale_body,
        grid=(nb,),
        in_specs=[pl.BlockSpec((NC, NP // nb, HID), lambda i: (0, i, 0)),
                  pl.BlockSpec((NP // nb, HID), lambda i: (i, 0))],
        out_specs=[pl.BlockSpec((NP // nb, HID), lambda i: (i, 0)),
                   pl.BlockSpec((NP // nb, 16), lambda i: (i, 0))],
        out_shape=[jax.ShapeDtypeStruct((NP, HID), jnp.float32),
                   jax.ShapeDtypeStruct((NP, 16), jnp.float32)],
    )(dega, xw1)


_NB = 16
_BLK = NP // _NB


def _blk_mask(i):
    ridx = i * _BLK + lax.broadcasted_iota(jnp.int32, (_BLK, 1), 0)
    return (ridx < N).astype(jnp.float32)


def _mid_stats_body(p_ref, y1_ref, dinv_ref, b1_ref, h_ref, sums_ref):
    i = pl.program_id(0)
    dinv = dinv_ref[...][:, 0:1]
    h = (p_ref[0] + p_ref[1] - y1_ref[...]) * dinv + b1_ref[...]
    h = jnp.maximum(h, 0.0)
    h_ref[...] = h
    hm = h * _blk_mask(i)

    @pl.when(i == 0)
    def _():
        sums_ref[...] = jnp.zeros_like(sums_ref)

    sums_ref[0:1, :] += jnp.sum(hm, axis=0, keepdims=True)
    sums_ref[1:2, :] += jnp.sum(hm * hm, axis=0, keepdims=True)


def _mid_emit_body(h_ref, dinv_ref, sums_ref, g1_ref, bt1_ref, w2_ref, o_ref):
    m = sums_ref[0:1, :] / N
    v = sums_ref[1:2, :] / N - m * m
    hn = (h_ref[...] - m) / jnp.sqrt(v + 1e-5) * g1_ref[...] + bt1_ref[...]
    y2 = jnp.dot(hn, w2_ref[...], **_DOT) * dinv_ref[...][:, 0:1]
    o_ref[0] = y2[:, 0:HID]
    o_ref[1] = y2[:, HID:OUT]


def _tc_mid(p, y1, dinv16, b1, g1, beta1, w2):
    h, sums = pl.pallas_call(
        _mid_stats_body,
        grid=(_NB,),
        in_specs=[pl.BlockSpec((NC, _BLK, HID), lambda i: (0, i, 0)),
                  pl.BlockSpec((_BLK, HID), lambda i: (i, 0)),
                  pl.BlockSpec((_BLK, 16), lambda i: (i, 0)),
                  pl.BlockSpec((1, HID), lambda i: (0, 0))],
        out_specs=[pl.BlockSpec((_BLK, HID), lambda i: (i, 0)),
                   pl.BlockSpec((2, HID), lambda i: (0, 0))],
        out_shape=[jax.ShapeDtypeStruct((NP, HID), jnp.float32),
                   jax.ShapeDtypeStruct((2, HID), jnp.float32)],
    )(p, y1, dinv16, b1)
    y2 = pl.pallas_call(
        _mid_emit_body,
        grid=(_NB,),
        in_specs=[pl.BlockSpec((_BLK, HID), lambda i: (i, 0)),
                  pl.BlockSpec((_BLK, 16), lambda i: (i, 0)),
                  pl.BlockSpec((2, HID), lambda i: (0, 0)),
                  pl.BlockSpec((1, HID), lambda i: (0, 0)),
                  pl.BlockSpec((1, HID), lambda i: (0, 0)),
                  pl.BlockSpec((HID, OUT), lambda i: (0, 0))],
        out_specs=pl.BlockSpec((2, _BLK, HID), lambda i: (0, i, 0)),
        out_shape=jax.ShapeDtypeStruct((2, NP, HID), jnp.float32),
    )(h, dinv16, sums, g1, beta1, w2)
    return y2.reshape(2 * NP, HID)


def _bn2_stats_body(z_ref, dinv_ref, b2_ref, x2_ref, sums_ref):
    i = pl.program_id(0)
    dinv = dinv_ref[...][:, 0:1]
    x = jnp.concatenate([z_ref[0], z_ref[1]], axis=1) * dinv + b2_ref[...]
    x = jnp.maximum(x, 0.0)
    x2_ref[...] = x
    xm = x * _blk_mask(i)

    @pl.when(i == 0)
    def _():
        sums_ref[...] = jnp.zeros_like(sums_ref)

    sums_ref[0:1, :] += jnp.sum(xm, axis=0, keepdims=True)
    sums_ref[1:2, :] += jnp.sum(xm * xm, axis=0, keepdims=True)


def _tc_bn2_stats(z, dinv16, b2):
    return pl.pallas_call(
        _bn2_stats_body,
        grid=(_NB,),
        in_specs=[pl.BlockSpec((NC, _BLK, HID), lambda i: (0, i, 0)),
                  pl.BlockSpec((_BLK, 16), lambda i: (i, 0)),
                  pl.BlockSpec((1, OUT), lambda i: (0, 0))],
        out_specs=[pl.BlockSpec((_BLK, OUT), lambda i: (i, 0)),
                   pl.BlockSpec((2, OUT), lambda i: (0, 0))],
        out_shape=[jax.ShapeDtypeStruct((NP, OUT), jnp.float32),
                   jax.ShapeDtypeStruct((2, OUT), jnp.float32)],
    )(z, dinv16, b2)


def _pool_body(x2_ref, sums_ref, g2_ref, bt2_ref, ib_ref, o_ref, x_ref):
    m = sums_ref[0:1, :] / N
    v = sums_ref[1:2, :] / N - m * m
    x_ref[...] = ((x2_ref[...] - m) / jnp.sqrt(v + 1e-5) * g2_ref[...]
                  + bt2_ref[...])

    neg_inf = jnp.float32(float("-inf"))

    def upper_bound(bval):
        def cond(st):
            return st[0] < st[1]

        def body(st):
            lo, hi = st
            mid = (lo + hi) // 2
            gt = ib_ref[mid] > bval
            return (jnp.where(gt, lo, mid + 1), jnp.where(gt, mid, hi))

        return lax.while_loop(cond, body, (jnp.int32(0), jnp.int32(N)))[0]

    def group(g, start):
        outs = []
        for j in range(8):
            b = g * 8 + j
            stop = upper_bound(b)

            def cond(st):
                return st[0] < stop

            def body(st):
                p, run = st
                rows = x_ref[pl.ds(pl.multiple_of(p, 8), 8), :]
                ridx = p + lax.broadcasted_iota(jnp.int32, (8, 1), 0)
                mask = (ridx >= start) & (ridx < stop)
                return p + 8, jnp.maximum(run, jnp.where(mask, rows, neg_inf))

            p0 = (start // 8) * 8
            run0 = jnp.full((8, OUT), neg_inf, dtype=jnp.float32)
            _, run = lax.while_loop(cond, body, (p0, run0))
            outs.append(jnp.max(run, axis=0, keepdims=True))
            start = stop
        o_ref[pl.ds(g * 8, 8), :] = jnp.concatenate(outs, axis=0)
        return start

    lax.fori_loop(0, B // 8, group, jnp.int32(0))


def _tc_pool(z, dinv16, b2, g2, beta2, ibatch):
    x2, sums = _tc_bn2_stats(z, dinv16, b2)
    return pl.pallas_call(
        _pool_body,
        in_specs=[
            pl.BlockSpec((NP, OUT), lambda: (0, 0)),
            pl.BlockSpec((2, OUT), lambda: (0, 0)),
            pl.BlockSpec((1, OUT), lambda: (0, 0)),
            pl.BlockSpec((1, OUT), lambda: (0, 0)),
            pl.BlockSpec(memory_space=pltpu.SMEM),
        ],
        out_shape=jax.ShapeDtypeStruct((B, OUT), jnp.float32),
        scratch_shapes=[pltpu.VMEM((NP, OUT), jnp.float32)],
    )(x2, sums, g2, beta2, ibatch)


def _cell_body(g_ref, wc1_ref, bc1_ref, gc1_ref, btc1_ref, wc2_ref, bc2_ref,
               o_ref):
    t = jnp.tanh(jnp.dot(g_ref[...], wc1_ref[...], **_DOT) + bc1_ref[...])
    m = jnp.mean(t, axis=0, keepdims=True)
    v = jnp.mean((t - m) ** 2, axis=0, keepdims=True)
    tn = (t - m) / jnp.sqrt(v + 1e-5) * gc1_ref[...] + btc1_ref[...]
    o_ref[...] = jnp.maximum(jnp.dot(tn, wc2_ref[...], **_DOT) + bc2_ref[...],
                             0.0)


def _tc_cell(gexpr, wc1, bc1, gc1, betac1, wc2, bc2):
    return pl.pallas_call(
        _cell_body,
        out_shape=jax.ShapeDtypeStruct((B, OUT), jnp.float32),
    )(gexpr, wc1, bc1, gc1, betac1, wc2, bc2)


def kernel(drug_feature, drug_adj, ibatch, gexpr_data, W1, b1, g1, beta1,
           W2, b2, g2, beta2, Wc1, bc1, gc1, betac1, Wc2, bc2):
    src = drug_adj[0].astype(jnp.int32)
    dst = drug_adj[1].astype(jnp.int32)
    src2 = jnp.concatenate([src, src + jnp.int32(NP)])

    x_pad = jnp.pad(drug_feature, ((0, NP - N), (0, 0)))

    zeros_init = jnp.zeros((RPS, HID), jnp.float32)
    ones_rows = jnp.ones((CH, HID), jnp.float32)

    dega = _sc_degree(zeros_init, ones_rows, dst)
    xw1 = _tc_xw1(x_pad, W1)
    y1, dinv16 = _tc_scale(dega, xw1)

    p1 = _sc_conv(y1, src, dst,
                  edge_core_stride=E // NC, per_core_edges=E // NC,
                  idx_core_stride=0, init_core_stride=0)

    x_cell = _tc_cell(gexpr_data, Wc1, bc1.reshape(1, HID),
                      gc1.reshape(1, HID), betac1.reshape(1, HID),
                      Wc2, bc2.reshape(1, OUT))

    y2cat = _tc_mid(p1, y1, dinv16, b1.reshape(1, HID), g1.reshape(1, HID),
                    beta1.reshape(1, HID), W2)

    z = _sc_conv(y2cat, src2, dst,
                 edge_core_stride=0, per_core_edges=E,
                 idx_core_stride=E, init_core_stride=NP)

    x_drug = _tc_pool(z, dinv16, b2.reshape(1, OUT), g2.reshape(1, OUT),
                      beta2.reshape(1, OUT), ibatch.astype(jnp.int32))

    return (x_drug, x_cell)


# speedup vs baseline: 23.2557x; 2.4634x over previous
import functools

import jax
import jax.numpy as jnp
from jax import lax
from jax.experimental import pallas as pl
from jax.experimental.pallas import tpu as pltpu
from jax.experimental.pallas import tpu_sc as plsc

N = 10000
E = 320000
D_DRUG = 128
D_CELL = 954
HID = 128
OUT = 256
B = 128

NC = 2
NS = 16
CH = 80
NP = 10240
RPS = NP // NS

@functools.cache
def _mesh():
    return plsc.VectorSubcoreMesh(core_axis_name="c", subcore_axis_name="s")


G = 4
NBUF = 8


def _sc_conv(y, src4, dst_idx, *, dst_per_core, nchunks, init_core_stride):
    nsteps = -(-nchunks // NBUF) * NBUF

    @functools.partial(
        pl.kernel,
        out_type=jax.ShapeDtypeStruct((NC, NP, HID), jnp.float32),
        mesh=_mesh(),
        scratch_types=(
            [pltpu.VMEM_SHARED((NP, HID), jnp.float32),
             pltpu.VMEM((NBUF, CH), jnp.int32),
             pltpu.VMEM((NBUF, CH), jnp.int32)]
            + [pltpu.VMEM((CH, HID), jnp.float32) for _ in range(G)]
            + [pltpu.SemaphoreType.DMA for _ in range(NBUF + G)]
        ),
    )
    def k(y_hbm, src_hbm, dst_hbm, out_hbm, acc_sh, sidx, didx, *rest):
        rows = rest[:G]
        isems = rest[G:G + NBUF]
        gsems = rest[G + NBUF:]
        c = lax.axis_index("c")
        s = lax.axis_index("s")
        r0 = s * RPS

        def src_row(kk):
            return src_hbm.at[c].at[s].at[kk]

        def dst_row(kk):
            if dst_per_core:
                return dst_hbm.at[c].at[s].at[kk]
            return dst_hbm.at[s].at[kk]

        pltpu.sync_copy(y_hbm.at[pl.ds(init_core_stride * c + r0, RPS)],
                        acc_sh.at[pl.ds(r0, RPS)])
        plsc.subcore_barrier()

        for j in range(NBUF):
            pltpu.async_copy(src_row(j), sidx.at[j], isems[j])
            pltpu.async_copy(dst_row(j), didx.at[j], isems[j])
        for b in range(G):
            pltpu.make_async_copy(src_row(b), sidx.at[b], isems[b]).wait()
            pltpu.make_async_copy(dst_row(b), didx.at[b], isems[b]).wait()
            pltpu.async_copy(y_hbm.at[sidx.at[b]], rows[b], gsems[b])

        @pl.loop(0, nsteps, step=NBUF)
        def _(o):
            for j in range(NBUF):
                kk = o + j
                b = j % G
                ji = (j + G) % NBUF

                @pl.when(kk < nchunks)
                def _():
                    pltpu.make_async_copy(y_hbm.at[sidx.at[j]], rows[b],
                                          gsems[b]).wait()
                    pltpu.sync_copy(rows[b], acc_sh.at[didx.at[j]], add=True)

                @pl.when(kk + NBUF < nchunks)
                def _():
                    pltpu.async_copy(src_row(kk + NBUF), sidx.at[j], isems[j])
                    pltpu.async_copy(dst_row(kk + NBUF), didx.at[j], isems[j])

                @pl.when(kk + G < nchunks)
                def _():
                    pltpu.make_async_copy(src_row(kk + G), sidx.at[ji],
                                          isems[ji]).wait()
                    pltpu.make_async_copy(dst_row(kk + G), didx.at[ji],
                                          isems[ji]).wait()
                    pltpu.async_copy(y_hbm.at[sidx.at[ji]], rows[b], gsems[b])

        plsc.subcore_barrier()
        pltpu.sync_copy(acc_sh.at[pl.ds(r0, RPS)],
                        out_hbm.at[c].at[pl.ds(r0, RPS)])

    return k(y, src4, dst_idx)


def _sc_degree(zeros_init, ones_rows, dst4):
    nchunks = E // NC // NS // CH

    @functools.partial(
        pl.kernel,
        out_type=jax.ShapeDtypeStruct((NC, NP, HID), jnp.float32),
        mesh=_mesh(),
        scratch_types=[
            pltpu.VMEM_SHARED((NP, HID), jnp.float32),
            pltpu.VMEM((nchunks, CH), jnp.int32),
            pltpu.VMEM((CH, HID), jnp.float32),
        ],
    )
    def k(z_hbm, ones_hbm, dst_hbm, out_hbm, acc_sh, didx, ones_v):
        c = lax.axis_index("c")
        s = lax.axis_index("s")
        r0 = s * RPS
        pltpu.sync_copy(z_hbm, acc_sh.at[pl.ds(r0, RPS)])
        pltpu.sync_copy(ones_hbm, ones_v)
        pltpu.sync_copy(dst_hbm.at[c].at[s], didx)
        plsc.subcore_barrier()

        @pl.loop(0, nchunks)
        def _(kk):
            pltpu.sync_copy(ones_v, acc_sh.at[didx.at[kk]], add=True)

        plsc.subcore_barrier()
        pltpu.sync_copy(acc_sh.at[pl.ds(r0, RPS)],
                        out_hbm.at[c].at[pl.ds(r0, RPS)])

    return k(zeros_init, ones_rows, dst4)


_DOT = dict(preferred_element_type=jnp.float32, precision=lax.Precision.HIGHEST)


def _mm_body(x_ref, w_ref, o_ref):
    o_ref[...] = jnp.dot(x_ref[...], w_ref[...], **_DOT)


def _tc_xw1(x, w):
    nb = 16
    return pl.pallas_call(
        _mm_body,
        grid=(nb,),
        in_specs=[pl.BlockSpec((NP // nb, D_DRUG), lambda i: (i, 0)),
                  pl.BlockSpec((D_DRUG, HID), lambda i: (0, 0))],
        out_specs=pl.BlockSpec((NP // nb, HID), lambda i: (i, 0)),
        out_shape=jax.ShapeDtypeStruct((NP, HID), jnp.float32),
    )(x, w)


def _scale_body(dega_ref, xw_ref, y_ref, dinv_ref):
    deg = dega_ref[0] + dega_ref[1] + 1.0
    dinv = lax.rsqrt(jnp.maximum(deg, 1e-12))
    dinv_ref[...] = dinv[:, 0:16]
    y_ref[...] = xw_ref[...] * dinv[:, 0:1]


def _tc_scale(dega, xw1):
    nb = 16
    return pl.pallas_call(
        _scale_body,
        grid=(nb,),
        in_specs=[pl.BlockSpec((NC, NP // nb, HID), lambda i: (0, i, 0)),
                  pl.BlockSpec((NP // nb, HID), lambda i: (i, 0))],
        out_specs=[pl.BlockSpec((NP // nb, HID), lambda i: (i, 0)),
                   pl.BlockSpec((NP // nb, 16), lambda i: (i, 0))],
        out_shape=[jax.ShapeDtypeStruct((NP, HID), jnp.float32),
                   jax.ShapeDtypeStruct((NP, 16), jnp.float32)],
    )(dega, xw1)


_NB = 16
_BLK = NP // _NB


def _blk_mask(i):
    ridx = i * _BLK + lax.broadcasted_iota(jnp.int32, (_BLK, 1), 0)
    return (ridx < N).astype(jnp.float32)


def _mid_stats_body(p_ref, y1_ref, dinv_ref, b1_ref, h_ref, sums_ref):
    i = pl.program_id(0)
    dinv = dinv_ref[...][:, 0:1]
    h = (p_ref[0] + p_ref[1] - y1_ref[...]) * dinv + b1_ref[...]
    h = jnp.maximum(h, 0.0)
    h_ref[...] = h
    hm = h * _blk_mask(i)

    @pl.when(i == 0)
    def _():
        sums_ref[...] = jnp.zeros_like(sums_ref)

    sums_ref[0:1, :] += jnp.sum(hm, axis=0, keepdims=True)
    sums_ref[1:2, :] += jnp.sum(hm * hm, axis=0, keepdims=True)


def _mid_emit_body(h_ref, dinv_ref, sums_ref, g1_ref, bt1_ref, w2_ref, o_ref):
    m = sums_ref[0:1, :] / N
    v = sums_ref[1:2, :] / N - m * m
    hn = (h_ref[...] - m) / jnp.sqrt(v + 1e-5) * g1_ref[...] + bt1_ref[...]
    y2 = jnp.dot(hn, w2_ref[...], **_DOT) * dinv_ref[...][:, 0:1]
    o_ref[0] = y2[:, 0:HID]
    o_ref[1] = y2[:, HID:OUT]


def _tc_mid(p, y1, dinv16, b1, g1, beta1, w2):
    h, sums = pl.pallas_call(
        _mid_stats_body,
        grid=(_NB,),
        in_specs=[pl.BlockSpec((NC, _BLK, HID), lambda i: (0, i, 0)),
                  pl.BlockSpec((_BLK, HID), lambda i: (i, 0)),
                  pl.BlockSpec((_BLK, 16), lambda i: (i, 0)),
                  pl.BlockSpec((1, HID), lambda i: (0, 0))],
        out_specs=[pl.BlockSpec((_BLK, HID), lambda i: (i, 0)),
                   pl.BlockSpec((2, HID), lambda i: (0, 0))],
        out_shape=[jax.ShapeDtypeStruct((NP, HID), jnp.float32),
                   jax.ShapeDtypeStruct((2, HID), jnp.float32)],
    )(p, y1, dinv16, b1)
    y2 = pl.pallas_call(
        _mid_emit_body,
        grid=(_NB,),
        in_specs=[pl.BlockSpec((_BLK, HID), lambda i: (i, 0)),
                  pl.BlockSpec((_BLK, 16), lambda i: (i, 0)),
                  pl.BlockSpec((2, HID), lambda i: (0, 0)),
                  pl.BlockSpec((1, HID), lambda i: (0, 0)),
                  pl.BlockSpec((1, HID), lambda i: (0, 0)),
                  pl.BlockSpec((HID, OUT), lambda i: (0, 0))],
        out_specs=pl.BlockSpec((2, _BLK, HID), lambda i: (0, i, 0)),
        out_shape=jax.ShapeDtypeStruct((2, NP, HID), jnp.float32),
    )(h, dinv16, sums, g1, beta1, w2)
    return y2.reshape(2 * NP, HID)


def _bn2_stats_body(z_ref, dinv_ref, b2_ref, x2_ref, sums_ref):
    i = pl.program_id(0)
    dinv = dinv_ref[...][:, 0:1]
    x = jnp.concatenate([z_ref[0], z_ref[1]], axis=1) * dinv + b2_ref[...]
    x = jnp.maximum(x, 0.0)
    x2_ref[...] = x
    xm = x * _blk_mask(i)

    @pl.when(i == 0)
    def _():
        sums_ref[...] = jnp.zeros_like(sums_ref)

    sums_ref[0:1, :] += jnp.sum(xm, axis=0, keepdims=True)
    sums_ref[1:2, :] += jnp.sum(xm * xm, axis=0, keepdims=True)


def _tc_bn2_stats(z, dinv16, b2):
    return pl.pallas_call(
        _bn2_stats_body,
        grid=(_NB,),
        in_specs=[pl.BlockSpec((NC, _BLK, HID), lambda i: (0, i, 0)),
                  pl.BlockSpec((_BLK, 16), lambda i: (i, 0)),
                  pl.BlockSpec((1, OUT), lambda i: (0, 0))],
        out_specs=[pl.BlockSpec((_BLK, OUT), lambda i: (i, 0)),
                   pl.BlockSpec((2, OUT), lambda i: (0, 0))],
        out_shape=[jax.ShapeDtypeStruct((NP, OUT), jnp.float32),
                   jax.ShapeDtypeStruct((2, OUT), jnp.float32)],
    )(z, dinv16, b2)


def _pool_body(x2_ref, sums_ref, g2_ref, bt2_ref, ib_ref, o_ref, x_ref):
    m = sums_ref[0:1, :] / N
    v = sums_ref[1:2, :] / N - m * m
    x_ref[...] = ((x2_ref[...] - m) / jnp.sqrt(v + 1e-5) * g2_ref[...]
                  + bt2_ref[...])

    neg_inf = jnp.float32(float("-inf"))

    def upper_bound(bval):
        def cond(st):
            return st[0] < st[1]

        def body(st):
            lo, hi = st
            mid = (lo + hi) // 2
            gt = ib_ref[mid] > bval
            return (jnp.where(gt, lo, mid + 1), jnp.where(gt, mid, hi))

        return lax.while_loop(cond, body, (jnp.int32(0), jnp.int32(N)))[0]

    def group(g, start):
        outs = []
        for j in range(8):
            b = g * 8 + j
            stop = upper_bound(b)

            def cond(st):
                return st[0] < stop

            def body(st):
                p, run = st
                rows = x_ref[pl.ds(pl.multiple_of(p, 8), 8), :]
                ridx = p + lax.broadcasted_iota(jnp.int32, (8, 1), 0)
                mask = (ridx >= start) & (ridx < stop)
                return p + 8, jnp.maximum(run, jnp.where(mask, rows, neg_inf))

            p0 = (start // 8) * 8
            run0 = jnp.full((8, OUT), neg_inf, dtype=jnp.float32)
            _, run = lax.while_loop(cond, body, (p0, run0))
            outs.append(jnp.max(run, axis=0, keepdims=True))
            start = stop
        o_ref[pl.ds(g * 8, 8), :] = jnp.concatenate(outs, axis=0)
        return start

    lax.fori_loop(0, B // 8, group, jnp.int32(0))


def _tc_pool(z, dinv16, b2, g2, beta2, ibatch):
    x2, sums = _tc_bn2_stats(z, dinv16, b2)
    return pl.pallas_call(
        _pool_body,
        in_specs=[
            pl.BlockSpec((NP, OUT), lambda: (0, 0)),
            pl.BlockSpec((2, OUT), lambda: (0, 0)),
            pl.BlockSpec((1, OUT), lambda: (0, 0)),
            pl.BlockSpec((1, OUT), lambda: (0, 0)),
            pl.BlockSpec(memory_space=pltpu.SMEM),
        ],
        out_shape=jax.ShapeDtypeStruct((B, OUT), jnp.float32),
        scratch_shapes=[pltpu.VMEM((NP, OUT), jnp.float32)],
    )(x2, sums, g2, beta2, ibatch)


def _cell_body(g_ref, wc1_ref, bc1_ref, gc1_ref, btc1_ref, wc2_ref, bc2_ref,
               o_ref):
    t = jnp.tanh(jnp.dot(g_ref[...], wc1_ref[...], **_DOT) + bc1_ref[...])
    m = jnp.mean(t, axis=0, keepdims=True)
    v = jnp.mean((t - m) ** 2, axis=0, keepdims=True)
    tn = (t - m) / jnp.sqrt(v + 1e-5) * gc1_ref[...] + btc1_ref[...]
    o_ref[...] = jnp.maximum(jnp.dot(tn, wc2_ref[...], **_DOT) + bc2_ref[...],
                             0.0)


def _tc_cell(gexpr, wc1, bc1, gc1, betac1, wc2, bc2):
    return pl.pallas_call(
        _cell_body,
        out_shape=jax.ShapeDtypeStruct((B, OUT), jnp.float32),
    )(gexpr, wc1, bc1, gc1, betac1, wc2, bc2)


def kernel(drug_feature, drug_adj, ibatch, gexpr_data, W1, b1, g1, beta1,
           W2, b2, g2, beta2, Wc1, bc1, gc1, betac1, Wc2, bc2):
    src = drug_adj[0].astype(jnp.int32)
    dst = drug_adj[1].astype(jnp.int32)
    src2 = jnp.concatenate([src, src + jnp.int32(NP)])

    nch1 = E // NC // NS // CH
    nch2 = E // NS // CH
    src4_1 = src.reshape(NC, NS, nch1, CH)
    dst4_1 = dst.reshape(NC, NS, nch1, CH)
    src4_2 = src2.reshape(NC, NS, nch2, CH)
    dst3_2 = dst.reshape(NS, nch2, CH)

    x_pad = jnp.pad(drug_feature, ((0, NP - N), (0, 0)))

    zeros_init = jnp.zeros((RPS, HID), jnp.float32)
    ones_rows = jnp.ones((CH, HID), jnp.float32)

    dega = _sc_degree(zeros_init, ones_rows, dst4_1)
    xw1 = _tc_xw1(x_pad, W1)
    y1, dinv16 = _tc_scale(dega, xw1)

    p1 = _sc_conv(y1, src4_1, dst4_1, dst_per_core=True, nchunks=nch1,
                  init_core_stride=0)

    x_cell = _tc_cell(gexpr_data, Wc1, bc1.reshape(1, HID),
                      gc1.reshape(1, HID), betac1.reshape(1, HID),
                      Wc2, bc2.reshape(1, OUT))

    y2cat = _tc_mid(p1, y1, dinv16, b1.reshape(1, HID), g1.reshape(1, HID),
                    beta1.reshape(1, HID), W2)

    z = _sc_conv(y2cat, src4_2, dst3_2, dst_per_core=False, nchunks=nch2,
                 init_core_stride=NP)

    x_drug = _tc_pool(z, dinv16, b2.reshape(1, OUT), g2.reshape(1, OUT),
                      beta2.reshape(1, OUT), ibatch.astype(jnp.int32))

    return (x_drug, x_cell)
